# baseline (speedup 1.0000x reference)
import numpy as np
import jax
import jax.numpy as jnp
from jax.experimental import pallas as pl
from jax.experimental.pallas import tpu as pltpu

DIM = 1024
EMBES = 32
NUM_OPTIONS = 200
GROUP = EMBES + 1
PREFIX = 2 * EMBES + 4
TOTAL = PREFIX + NUM_OPTIONS * GROUP
GPB = 8
ROWS = GPB * GROUP
NSTEPS = TOTAL // ROWS + 1
EPS = 1e-12


def _ln(x, w, b):
    inv = 1.0 / DIM
    mu = jnp.sum(x, axis=-1, keepdims=True) * inv
    ex2 = jnp.sum(x * x, axis=-1, keepdims=True) * inv
    var = ex2 - mu * mu
    return (x - mu) * jax.lax.rsqrt(var + EPS) * w + b


def _emb_kernel(opts_ref, special_ref, pre_ref, w_ref, b_ref, pos_ref,
                out_ref, x_buf, carry):
    k = pl.program_id(0)
    ent = special_ref[1:2, :]
    head = GROUP * (GPB - 1) + 2

    @pl.when(k == 0)
    def _():
        x_buf[pl.ds(0, 1), :] = special_ref[0:1, :]
        x_buf[pl.ds(1, 1), :] = ent
        x_buf[pl.ds(2, EMBES), :] = pre_ref[0, 0]
        x_buf[pl.ds(2 + EMBES, 1), :] = special_ref[2:3, :]
        x_buf[pl.ds(3 + EMBES, EMBES), :] = pre_ref[0, 1]
        x_buf[pl.ds(PREFIX - 1, 1), :] = special_ref[3:4, :]
        for g in range(GPB - 3):
            base = PREFIX + GROUP * g
            x_buf[pl.ds(base, 1), :] = ent
            x_buf[pl.ds(base + 1, EMBES), :] = opts_ref[0, g]
        x_buf[pl.ds(head, 1), :] = ent
        x_buf[pl.ds(head + 1, 30), :] = opts_ref[0, GPB - 3][0:30]

    @pl.when(jnp.logical_and(k > 0, k < NSTEPS - 1))
    def _():
        x_buf[pl.ds(0, 2), :] = carry[pl.ds(30, 2), :]
        for gg in range(GPB - 1):
            base = 2 + GROUP * gg
            x_buf[pl.ds(base, 1), :] = ent
            if gg == 0:
                x_buf[pl.ds(base + 1, EMBES), :] = carry[pl.ds(32, EMBES), :]
            elif gg == 1:
                x_buf[pl.ds(base + 1, EMBES), :] = carry[pl.ds(64, EMBES), :]
            else:
                x_buf[pl.ds(base + 1, EMBES), :] = opts_ref[0, gg - 2]
        x_buf[pl.ds(head, 1), :] = ent
        x_buf[pl.ds(head + 1, 30), :] = opts_ref[0, GPB - 3][0:30]

    @pl.when(k == NSTEPS - 1)
    def _():
        x_buf[pl.ds(0, 2), :] = opts_ref[0, GPB - 3][30:32]
        x_buf[pl.ds(2, 1), :] = ent
        x_buf[pl.ds(3, EMBES), :] = opts_ref[0, GPB - 2]
        x_buf[pl.ds(3 + EMBES, 1), :] = ent
        x_buf[pl.ds(4 + EMBES, EMBES), :] = opts_ref[0, GPB - 1]

    @pl.when(k < NSTEPS - 2)
    def _():
        carry[pl.ds(0, EMBES), :] = opts_ref[0, GPB - 3]
        carry[pl.ds(EMBES, EMBES), :] = opts_ref[0, GPB - 2]
        carry[pl.ds(2 * EMBES, EMBES), :] = opts_ref[0, GPB - 1]

    x = x_buf[:, :] + pos_ref[:, :]
    y = _ln(x, w_ref[:], b_ref[:])
    out_ref[:, :] = y.reshape(ROWS * 8, 128)


def kernel(pre_embs, options_embs, special_table, pos_table, ln_w, ln_b):
    embeddings = pl.pallas_call(
        _emb_kernel,
        grid=(NSTEPS,),
        in_specs=[
            pl.BlockSpec((1, GPB, EMBES, DIM),
                         lambda k: (0, jnp.minimum(k, NSTEPS - 2), 0, 0)),
            pl.BlockSpec((4, DIM), lambda k: (0, 0)),
            pl.BlockSpec((1, 2, EMBES, DIM), lambda k: (0, 0, 0, 0)),
            pl.BlockSpec((DIM,), lambda k: (0,)),
            pl.BlockSpec((DIM,), lambda k: (0,)),
            pl.BlockSpec((ROWS, DIM), lambda k: (k, 0)),
        ],
        out_specs=pl.BlockSpec((ROWS * 8, 128), lambda k: (k, 0)),
        out_shape=jax.ShapeDtypeStruct((TOTAL * 8, 128), jnp.float32),
        scratch_shapes=[
            pltpu.VMEM((ROWS, DIM), jnp.float32),
            pltpu.VMEM((3 * EMBES, DIM), jnp.float32),
        ],
        compiler_params=pltpu.CompilerParams(
            dimension_semantics=("arbitrary",)),
    )(options_embs, special_table, pre_embs, ln_w, ln_b, pos_table)
    embeddings = embeddings.reshape(1, TOTAL, DIM)

    opt_pos = np.arange(PREFIX, PREFIX + NUM_OPTIONS * GROUP,
                        dtype=np.int64).reshape(NUM_OPTIONS, GROUP)
    opt_pos_ids = jnp.asarray(opt_pos)
    return embeddings, opt_pos_ids


# speedup vs baseline: 1.1740x; 1.1740x over previous
import numpy as np
import jax
import jax.numpy as jnp
from jax.experimental import pallas as pl
from jax.experimental.pallas import tpu as pltpu

DIM = 1024
EMBES = 32
NUM_OPTIONS = 200
GROUP = EMBES + 1
PREFIX = 2 * EMBES + 4
TOTAL = PREFIX + NUM_OPTIONS * GROUP
GPB = 40
ROWS = GPB * GROUP
NSTEPS = TOTAL // ROWS + 1
EPS = 1e-12


def _ln(x, w, b):
    inv = 1.0 / DIM
    mu = jnp.sum(x, axis=-1, keepdims=True) * inv
    ex2 = jnp.sum(x * x, axis=-1, keepdims=True) * inv
    var = ex2 - mu * mu
    return (x - mu) * jax.lax.rsqrt(var + EPS) * w + b


def _emb_kernel(opts_ref, special_ref, pre_ref, w_ref, b_ref, pos_ref,
                out_ref, x_buf, carry):
    k = pl.program_id(0)
    ent = special_ref[1:2, :]
    head = GROUP * (GPB - 1) + 2

    @pl.when(k == 0)
    def _():
        x_buf[pl.ds(0, 1), :] = special_ref[0:1, :]
        x_buf[pl.ds(1, 1), :] = ent
        x_buf[pl.ds(2, EMBES), :] = pre_ref[0, 0]
        x_buf[pl.ds(2 + EMBES, 1), :] = special_ref[2:3, :]
        x_buf[pl.ds(3 + EMBES, EMBES), :] = pre_ref[0, 1]
        x_buf[pl.ds(PREFIX - 1, 1), :] = special_ref[3:4, :]
        for g in range(GPB - 3):
            base = PREFIX + GROUP * g
            x_buf[pl.ds(base, 1), :] = ent
            x_buf[pl.ds(base + 1, EMBES), :] = opts_ref[0, g]
        x_buf[pl.ds(head, 1), :] = ent
        x_buf[pl.ds(head + 1, 30), :] = opts_ref[0, GPB - 3][0:30]

    @pl.when(jnp.logical_and(k > 0, k < NSTEPS - 1))
    def _():
        x_buf[pl.ds(0, 2), :] = carry[pl.ds(30, 2), :]
        for gg in range(GPB - 1):
            base = 2 + GROUP * gg
            x_buf[pl.ds(base, 1), :] = ent
            if gg == 0:
                x_buf[pl.ds(base + 1, EMBES), :] = carry[pl.ds(32, EMBES), :]
            elif gg == 1:
                x_buf[pl.ds(base + 1, EMBES), :] = carry[pl.ds(64, EMBES), :]
            else:
                x_buf[pl.ds(base + 1, EMBES), :] = opts_ref[0, gg - 2]
        x_buf[pl.ds(head, 1), :] = ent
        x_buf[pl.ds(head + 1, 30), :] = opts_ref[0, GPB - 3][0:30]

    @pl.when(k == NSTEPS - 1)
    def _():
        x_buf[pl.ds(0, 2), :] = opts_ref[0, GPB - 3][30:32]
        x_buf[pl.ds(2, 1), :] = ent
        x_buf[pl.ds(3, EMBES), :] = opts_ref[0, GPB - 2]
        x_buf[pl.ds(3 + EMBES, 1), :] = ent
        x_buf[pl.ds(4 + EMBES, EMBES), :] = opts_ref[0, GPB - 1]

    @pl.when(k < NSTEPS - 2)
    def _():
        carry[pl.ds(0, EMBES), :] = opts_ref[0, GPB - 3]
        carry[pl.ds(EMBES, EMBES), :] = opts_ref[0, GPB - 2]
        carry[pl.ds(2 * EMBES, EMBES), :] = opts_ref[0, GPB - 1]

    x = x_buf[:, :] + pos_ref[:, :]
    y = _ln(x, w_ref[:], b_ref[:])
    out_ref[:, :] = y.reshape(ROWS * 8, 128)


def kernel(pre_embs, options_embs, special_table, pos_table, ln_w, ln_b):
    embeddings = pl.pallas_call(
        _emb_kernel,
        grid=(NSTEPS,),
        in_specs=[
            pl.BlockSpec((1, GPB, EMBES, DIM),
                         lambda k: (0, jnp.minimum(k, NSTEPS - 2), 0, 0)),
            pl.BlockSpec((4, DIM), lambda k: (0, 0)),
            pl.BlockSpec((1, 2, EMBES, DIM), lambda k: (0, 0, 0, 0)),
            pl.BlockSpec((DIM,), lambda k: (0,)),
            pl.BlockSpec((DIM,), lambda k: (0,)),
            pl.BlockSpec((ROWS, DIM), lambda k: (k, 0)),
        ],
        out_specs=pl.BlockSpec((ROWS * 8, 128), lambda k: (k, 0)),
        out_shape=jax.ShapeDtypeStruct((TOTAL * 8, 128), jnp.float32),
        scratch_shapes=[
            pltpu.VMEM((ROWS, DIM), jnp.float32),
            pltpu.VMEM((3 * EMBES, DIM), jnp.float32),
        ],
        compiler_params=pltpu.CompilerParams(
            dimension_semantics=("arbitrary",)),
    )(options_embs, special_table, pre_embs, ln_w, ln_b, pos_table)
    embeddings = embeddings.reshape(1, TOTAL, DIM)

    opt_pos = np.arange(PREFIX, PREFIX + NUM_OPTIONS * GROUP,
                        dtype=np.int64).reshape(NUM_OPTIONS, GROUP)
    opt_pos_ids = jnp.asarray(opt_pos)
    return embeddings, opt_pos_ids


# speedup vs baseline: 1.2012x; 1.0232x over previous
import numpy as np
import jax
import jax.numpy as jnp
from jax.experimental import pallas as pl
from jax.experimental.pallas import tpu as pltpu

DIM = 1024
EMBES = 32
NUM_OPTIONS = 200
GROUP = EMBES + 1
PREFIX = 2 * EMBES + 4
TOTAL = PREFIX + NUM_OPTIONS * GROUP
GPB = 40
ROWS = GPB * GROUP
NSTEPS = TOTAL // ROWS + 1
EPS = 1e-12


def _ln(x, w, b):
    inv = 1.0 / DIM
    mu = jnp.sum(x, axis=-1, keepdims=True) * inv
    ex2 = jnp.sum(x * x, axis=-1, keepdims=True) * inv
    var = ex2 - mu * mu
    return (x - mu) * jax.lax.rsqrt(var + EPS) * w + b


def _emb_kernel(opts_ref, special_ref, pre_ref, w_ref, b_ref, pos_ref,
                pos_tail_ref, out_ref, x_buf, carry):
    k = pl.program_id(0)
    ent = special_ref[1:2, :]
    head = GROUP * (GPB - 1) + 2

    @pl.when(k == 0)
    def _():
        x_buf[pl.ds(0, 1), :] = special_ref[0:1, :]
        x_buf[pl.ds(1, 1), :] = ent
        x_buf[pl.ds(2, EMBES), :] = pre_ref[0, 0]
        x_buf[pl.ds(2 + EMBES, 1), :] = special_ref[2:3, :]
        x_buf[pl.ds(3 + EMBES, EMBES), :] = pre_ref[0, 1]
        x_buf[pl.ds(PREFIX - 1, 1), :] = special_ref[3:4, :]
        for g in range(GPB - 3):
            base = PREFIX + GROUP * g
            x_buf[pl.ds(base, 1), :] = ent
            x_buf[pl.ds(base + 1, EMBES), :] = opts_ref[0, g]
        x_buf[pl.ds(head, 1), :] = ent
        x_buf[pl.ds(head + 1, 30), :] = opts_ref[0, GPB - 3][0:30]

    @pl.when(jnp.logical_and(k > 0, k < NSTEPS - 1))
    def _():
        x_buf[pl.ds(0, 2), :] = carry[pl.ds(30, 2), :]
        for gg in range(GPB - 1):
            base = 2 + GROUP * gg
            x_buf[pl.ds(base, 1), :] = ent
            if gg == 0:
                x_buf[pl.ds(base + 1, EMBES), :] = carry[pl.ds(32, EMBES), :]
            elif gg == 1:
                x_buf[pl.ds(base + 1, EMBES), :] = carry[pl.ds(64, EMBES), :]
            else:
                x_buf[pl.ds(base + 1, EMBES), :] = opts_ref[0, gg - 2]
        x_buf[pl.ds(head, 1), :] = ent
        x_buf[pl.ds(head + 1, 30), :] = opts_ref[0, GPB - 3][0:30]

    @pl.when(k == NSTEPS - 1)
    def _():
        x_buf[pl.ds(0, 2), :] = opts_ref[0, GPB - 3][30:32]
        x_buf[pl.ds(2, 1), :] = ent
        x_buf[pl.ds(3, EMBES), :] = opts_ref[0, GPB - 2]
        x_buf[pl.ds(3 + EMBES, 1), :] = ent
        x_buf[pl.ds(4 + EMBES, EMBES), :] = opts_ref[0, GPB - 1]

    @pl.when(k < NSTEPS - 2)
    def _():
        carry[pl.ds(0, EMBES), :] = opts_ref[0, GPB - 3]
        carry[pl.ds(EMBES, EMBES), :] = opts_ref[0, GPB - 2]
        carry[pl.ds(2 * EMBES, EMBES), :] = opts_ref[0, GPB - 1]

    @pl.when(k < NSTEPS - 1)
    def _():
        x = x_buf[:, :] + pos_ref[:, :]
        y = _ln(x, w_ref[:], b_ref[:])
        out_ref[:, :] = y.reshape(ROWS * 8, 128)

    @pl.when(k == NSTEPS - 1)
    def _():
        xt = x_buf[pl.ds(0, 72), :] + pos_tail_ref[pl.ds(0, 72), :]
        yt = _ln(xt, w_ref[:], b_ref[:])
        out_ref[pl.ds(0, 72 * 8), :] = yt.reshape(72 * 8, 128)


def kernel(pre_embs, options_embs, special_table, pos_table, ln_w, ln_b):
    embeddings = pl.pallas_call(
        _emb_kernel,
        grid=(NSTEPS,),
        in_specs=[
            pl.BlockSpec((1, GPB, EMBES, DIM),
                         lambda k: (0, jnp.minimum(k, NSTEPS - 2), 0, 0)),
            pl.BlockSpec((4, DIM), lambda k: (0, 0)),
            pl.BlockSpec((1, 2, EMBES, DIM), lambda k: (0, 0, 0, 0)),
            pl.BlockSpec((DIM,), lambda k: (0,)),
            pl.BlockSpec((DIM,), lambda k: (0,)),
            pl.BlockSpec((ROWS, DIM), lambda k: (jnp.minimum(k, NSTEPS - 2), 0)),
            pl.BlockSpec((88, DIM), lambda k: (TOTAL // ROWS * ROWS // 88, 0)),
        ],
        out_specs=pl.BlockSpec((ROWS * 8, 128), lambda k: (k, 0)),
        out_shape=jax.ShapeDtypeStruct((TOTAL * 8, 128), jnp.float32),
        scratch_shapes=[
            pltpu.VMEM((ROWS, DIM), jnp.float32),
            pltpu.VMEM((3 * EMBES, DIM), jnp.float32),
        ],
        compiler_params=pltpu.CompilerParams(
            dimension_semantics=("arbitrary",)),
    )(options_embs, special_table, pre_embs, ln_w, ln_b, pos_table,
      pos_table)
    embeddings = embeddings.reshape(1, TOTAL, DIM)

    opt_pos = np.arange(PREFIX, PREFIX + NUM_OPTIONS * GROUP,
                        dtype=np.int64).reshape(NUM_OPTIONS, GROUP)
    opt_pos_ids = jnp.asarray(opt_pos)
    return embeddings, opt_pos_ids
